# baseline (speedup 1.0000x reference)
import jax
import jax.numpy as jnp
from jax import lax
from jax.experimental import pallas as pl
from jax.experimental.pallas import tpu as pltpu
from jax.experimental.pallas import tpu_sc as plsc

_N_VARS = 100000
_N_CLAUSES = 100000
_E = 1600000
_P = 3.0
_A = 10.0

_NCORES = 1
_NSUB = 16
_NW = _NCORES * _NSUB
_L = 16

_C = 1024
_K = _C // 128
_E_PAD = 1638400
_NCHUNK = _E_PAD // (_NW * _C)
_NCH2 = 2 * _NCHUNK
_T = _C * _NCHUNK
_NBUF = 3

_NC_PAD = 100096
_SL = _NC_PAD // _NSUB
_R = _NC_PAD // 128
_PIECES = []
_off = 0
while _off < _SL:
    _PIECES.append((_off, min(_C, _SL - _off)))
    _off += _C


def _sc_kernel_body(xv_hbm, dst_all, src_all,
                    num0, den0, num1, den1,
                    xv_v,
                    dst_v0, dst_v1, dst_v2,
                    src_v0, src_v1, src_v2,
                    num_v0, num_v1, num_v2,
                    den_v0, den_v1, den_v2,
                    stage_v, acc_num, acc_den,
                    in_s0, in_s1, in_s2, sc_s0, sc_s1, sc_s2):
    cid = lax.axis_index("c")
    sid = lax.axis_index("s")
    wid = sid * _NCORES + cid
    dst_v = [dst_v0, dst_v1, dst_v2]
    src_v = [src_v0, src_v1, src_v2]
    num_v = [num_v0, num_v1, num_v2]
    den_v = [den_v0, den_v1, den_v2]
    in_sems = [in_s0, in_s1, in_s2]
    sc_sems = [sc_s0, sc_s1, sc_s2]

    def chunk_off(ii):
        off = jnp.where(ii < _NCHUNK,
                        wid * _T + ii * _C,
                        _E_PAD + wid * _T + (ii - _NCHUNK) * _C)
        return pl.multiple_of(off, _C)

    def fire_in(ii, b):
        off = chunk_off(ii)
        pltpu.async_copy(dst_all.at[pl.ds(pl.multiple_of(off // 128, _K), _K)],
                         dst_v[b], in_sems[b])
        pltpu.async_copy(src_all.at[pl.ds(off, _C)], src_v[b],
                         in_sems[b])

    def wait_in(ii, b):
        off = chunk_off(ii)
        pltpu.make_async_copy(
            dst_all.at[pl.ds(pl.multiple_of(off // 128, _K), _K)],
            dst_v[b], in_sems[b]).wait()
        pltpu.make_async_copy(src_all.at[pl.ds(off, _C)], src_v[b],
                              in_sems[b]).wait()

    def fire_scatters(b):
        for j in range(_K):
            sl = pl.ds(j * 128, 128)
            pltpu.async_copy(num_v[b].at[sl], acc_num.at[dst_v[b].at[j]],
                             sc_sems[b], add=True)
            pltpu.async_copy(den_v[b].at[sl], acc_den.at[dst_v[b].at[j]],
                             sc_sems[b], add=True)

    def drain_scatters(b):
        for j in range(_K):
            sl = pl.ds(j * 128, 128)
            pltpu.make_async_copy(num_v[b].at[sl],
                                  acc_num.at[dst_v[b].at[j]],
                                  sc_sems[b]).wait()
            pltpu.make_async_copy(den_v[b].at[sl],
                                  acc_den.at[dst_v[b].at[j]],
                                  sc_sems[b]).wait()

    def compute(ii, b):
        fneg = jnp.where(ii < _NCHUNK, 0.0, 1.0).astype(jnp.float32)
        c1 = 1.0 - 2.0 * fneg

        @plsc.parallel_loop(0, _C // _L, unroll=8)
        def _vec(j):
            sl = pl.ds(j * _L, _L)
            idx = src_v[b][sl]
            v = plsc.load_gather(xv_v, [idx])
            v = fneg + c1 * v
            e = jnp.exp(_P * v)
            num_v[b][sl] = v * e
            den_v[b][sl] = e

    fire_in(0, 0)
    pltpu.sync_copy(xv_hbm, xv_v)

    @pl.loop(0, _C // _L)
    def _zero(j):
        stage_v[pl.ds(j * _L, _L)] = jnp.zeros((_L,), jnp.float32)

    for acc in (acc_num, acc_den):
        for off, ln in _PIECES:
            pltpu.sync_copy(stage_v.at[pl.ds(0, ln)],
                            acc.at[pl.ds(sid * _SL + off, ln)])

    plsc.subcore_barrier()

    def do_chunk(ii, b, bn):
        wait_in(ii, b)

        @pl.when(ii >= 2)
        def _():
            drain_scatters(bn)

        @pl.when(ii <= _NCH2 - 2)
        def _():
            fire_in(ii + 1, bn)

        compute(ii, b)
        fire_scatters(b)

    main_end = (_NCH2 - 2) // _NBUF * _NBUF

    @pl.loop(0, main_end, step=_NBUF)
    def _main(i):
        for b in range(_NBUF):
            do_chunk(i + b, b, (b + 1) % _NBUF)

    for ii in range(main_end, _NCH2):
        do_chunk(jnp.int32(ii), ii % _NBUF, (ii + 1) % _NBUF)
    drain_scatters((_NCH2 - 2) % _NBUF)
    drain_scatters((_NCH2 - 1) % _NBUF)

    plsc.subcore_barrier()

    num_out = [num0, num1][:_NCORES]
    den_out = [den0, den1][:_NCORES]
    for core in range(_NCORES):
        @pl.when(cid == core)
        def _():
            for acc, out in ((acc_num, num_out[core]),
                             (acc_den, den_out[core])):
                for off, ln in _PIECES:
                    pltpu.sync_copy(acc.at[pl.ds(sid * _SL + off, ln)],
                                    stage_v.at[pl.ds(0, ln)])
                    pltpu.sync_copy(stage_v.at[pl.ds(0, ln)],
                                    out.at[pl.ds(sid * _SL + off, ln)])


def _make_sc_kernel():
    mesh = plsc.VectorSubcoreMesh(
        core_axis_name="c", subcore_axis_name="s",
        num_cores=_NCORES, num_subcores=_NSUB)
    out = jax.ShapeDtypeStruct((_NC_PAD,), jnp.float32)
    return pl.kernel(
        _sc_kernel_body,
        out_type=(out, out, out, out),
        mesh=mesh,
        compiler_params=pltpu.CompilerParams(needs_layout_passes=False),
        scratch_types=[
            pltpu.VMEM((_N_VARS,), jnp.float32),
            *([pltpu.VMEM((_K, 128), jnp.int32)] * _NBUF),
            *([pltpu.VMEM((_C,), jnp.int32)] * _NBUF),
            *([pltpu.VMEM((_C,), jnp.float32)] * _NBUF),
            *([pltpu.VMEM((_C,), jnp.float32)] * _NBUF),
            pltpu.VMEM((_C,), jnp.float32),
            pltpu.VMEM_SHARED((_NC_PAD,), jnp.float32),
            pltpu.VMEM_SHARED((_NC_PAD,), jnp.float32),
            pltpu.SemaphoreType.DMA,
            pltpu.SemaphoreType.DMA,
            pltpu.SemaphoreType.DMA,
            pltpu.SemaphoreType.DMA,
            pltpu.SemaphoreType.DMA,
            pltpu.SemaphoreType.DMA,
        ],
    )


def _fin_body(*refs):
    out_ref = refs[-1]
    cc_ref = refs[-2]
    half = (len(refs) - 2) // 2
    num = sum(r[...] for r in refs[:half])
    den = sum(r[...] for r in refs[half:2 * half])
    sm = 1.0 / (1.0 + jnp.exp(_A * (0.5 - num / den)))
    row = lax.broadcasted_iota(jnp.int32, (_R, 128), 0)
    col = lax.broadcasted_iota(jnp.int32, (_R, 128), 1)
    mask = (row * 128 + col) < _N_CLAUSES
    diff = jnp.where(mask, sm - cc_ref[...], 0.0)
    out_ref[0, 0] = jnp.sum(diff * diff) / _N_CLAUSES


def _finalize(ns, ds, cc_pad):
    shape2d = (_R, 128)
    args = [a.reshape(shape2d) for a in (*ns, *ds, cc_pad)]
    loss = pl.pallas_call(
        _fin_body,
        out_shape=jax.ShapeDtypeStruct((1, 1), jnp.float32),
        in_specs=[pl.BlockSpec(memory_space=pltpu.VMEM)] * len(args),
        out_specs=pl.BlockSpec(memory_space=pltpu.SMEM),
    )(*args)
    return loss[0, 0]


def kernel(xv, adj_pos, adj_neg, clause_count, is_train):
    del is_train
    xvf = xv.reshape(-1)
    padn = _E_PAD - _E
    pad_dst = jnp.full((padn,), _N_CLAUSES, jnp.int32)
    pad_src = jnp.zeros((padn,), jnp.int32)

    dst_all = jnp.concatenate(
        [adj_pos[0], pad_dst, adj_neg[0], pad_dst]).reshape(-1, 128)
    src_all = jnp.concatenate([adj_pos[1], pad_src, adj_neg[1], pad_src])

    outs = _make_sc_kernel()(xvf, dst_all, src_all)
    ns = [outs[2 * i] for i in range(_NCORES)]
    ds = [outs[2 * i + 1] for i in range(_NCORES)]

    cc_pad = jnp.pad(clause_count, (0, _NC_PAD - _N_CLAUSES))
    return _finalize(ns, ds, cc_pad)


# speedup vs baseline: 2.3983x; 2.3983x over previous
import jax
import jax.numpy as jnp
from jax import lax
from jax.experimental import pallas as pl
from jax.experimental.pallas import tpu as pltpu
from jax.experimental.pallas import tpu_sc as plsc

_N_VARS = 100000
_N_CLAUSES = 100000
_E = 1600000
_P = 3.0
_A = 10.0

_NCORES = 2
_NSUB = 16
_NW = _NCORES * _NSUB
_L = 16

_C = 1024
_K = _C // 128
_PT = _E // _NW
_NFULL = _PT // _C
_TAIL = _PT - _NFULL * _C
_TAILV = _TAIL // _L
_NBUF = 3

_NC_PAD = 100096
_SL = _NC_PAD // _NSUB
_R = _NC_PAD // 128
_PIECES = []
_off = 0
while _off < _SL:
    _PIECES.append((_off, min(_C, _SL - _off)))
    _off += _C


def _sc_kernel_body(xv_hbm, pos_flat, neg_flat,
                    num0, den0, num1, den1,
                    xv_v,
                    dst_v0, dst_v1, dst_v2,
                    src_v0, src_v1, src_v2,
                    num_v0, num_v1, num_v2,
                    den_v0, den_v1, den_v2,
                    stage_v, acc_num, acc_den,
                    in_s0, in_s1, in_s2, sc_s0, sc_s1, sc_s2):
    cid = lax.axis_index("c")
    sid = lax.axis_index("s")
    wid = sid * _NCORES + cid
    dst_v = [dst_v0, dst_v1, dst_v2]
    src_v = [src_v0, src_v1, src_v2]
    num_v = [num_v0, num_v1, num_v2]
    den_v = [den_v0, den_v1, den_v2]
    in_sems = [in_s0, in_s1, in_s2]
    sc_sems = [sc_s0, sc_s1, sc_s2]

    def in_copies(flat, jj, b, tail):
        off = pl.multiple_of(wid * _PT + jj * _C, 16)
        cps = []
        nrow = _TAIL // 128 if tail else _K
        for j in range(nrow):
            cps.append((flat.at[pl.ds(off + j * 128, 128)], dst_v[b].at[j]))
        if tail:
            cps.append((flat.at[pl.ds(off + nrow * 128, 64)],
                        dst_v[b].at[nrow, pl.ds(0, 64)]))
            cps.append((flat.at[pl.ds(_E + off, _TAIL)],
                        src_v[b].at[pl.ds(0, _TAIL)]))
        else:
            cps.append((flat.at[pl.ds(_E + off, _C)], src_v[b]))
        return cps

    def fire_in(flat, jj, b, tail=False):
        for s, d in in_copies(flat, jj, b, tail):
            pltpu.async_copy(s, d, in_sems[b])

    def wait_in(flat, jj, b, tail=False):
        for s, d in in_copies(flat, jj, b, tail):
            pltpu.make_async_copy(s, d, in_sems[b]).wait()

    def fire_scatters(b):
        for j in range(_K):
            sl = pl.ds(j * 128, 128)
            pltpu.async_copy(num_v[b].at[sl], acc_num.at[dst_v[b].at[j]],
                             sc_sems[b], add=True)
            pltpu.async_copy(den_v[b].at[sl], acc_den.at[dst_v[b].at[j]],
                             sc_sems[b], add=True)

    def drain_scatters(b):
        for j in range(_K):
            sl = pl.ds(j * 128, 128)
            pltpu.make_async_copy(num_v[b].at[sl],
                                  acc_num.at[dst_v[b].at[j]],
                                  sc_sems[b]).wait()
            pltpu.make_async_copy(den_v[b].at[sl],
                                  acc_den.at[dst_v[b].at[j]],
                                  sc_sems[b]).wait()

    def compute(b, is_neg, nvec, unroll=8):
        @plsc.parallel_loop(0, nvec, unroll=unroll)
        def _vec(j):
            sl = pl.ds(j * _L, _L)
            idx = src_v[b][sl]
            v = plsc.load_gather(xv_v, [idx])
            if is_neg:
                v = 1.0 - v
            e = jnp.exp(_P * v)
            num_v[b][sl] = v * e
            den_v[b][sl] = e

    def compute_tail(b, is_neg):
        compute(b, is_neg, _TAILV, unroll=4)
        zero = jnp.zeros((_L,), jnp.float32)
        for k in range(_TAILV, _C // _L):
            num_v[b][pl.ds(k * _L, _L)] = zero
            den_v[b][pl.ds(k * _L, _L)] = zero

    fire_in(pos_flat, 0, 0)
    pltpu.sync_copy(xv_hbm, xv_v)

    @pl.loop(0, _C // _L)
    def _zero(j):
        stage_v[pl.ds(j * _L, _L)] = jnp.zeros((_L,), jnp.float32)

    for acc in (acc_num, acc_den):
        for off, ln in _PIECES:
            pltpu.sync_copy(stage_v.at[pl.ds(0, ln)],
                            acc.at[pl.ds(sid * _SL + off, ln)])

    plsc.subcore_barrier()

    def do_chunk(flat, jj, b, bn, is_neg, nxt=None, drain=True,
                 drain_guard=None):
        wait_in(flat, jj, b)
        if drain:
            if drain_guard is not None:
                @pl.when(drain_guard)
                def _():
                    drain_scatters(bn)
            else:
                drain_scatters(bn)
        if nxt is not None:
            fire_in(flat, nxt, bn)
        compute(b, is_neg, _C // _L)
        fire_scatters(b)

    @pl.loop(0, 45, step=_NBUF)
    def _main1(i):
        for b in range(_NBUF):
            jj = i + b
            do_chunk(pos_flat, jj, b, (b + 1) % _NBUF, False, nxt=jj + 1,
                     drain_guard=jj >= 2)

    for ll in (45, 46):
        do_chunk(pos_flat, jnp.int32(ll), ll % _NBUF, (ll + 1) % _NBUF,
                 False, nxt=jnp.int32(ll + 1))
    do_chunk(pos_flat, jnp.int32(47), 2, 0, False, nxt=None)
    fire_in(pos_flat, jnp.int32(_NFULL), 0, tail=True)

    wait_in(pos_flat, jnp.int32(_NFULL), 0, tail=True)
    drain_scatters(1)
    fire_in(neg_flat, jnp.int32(0), 1)
    compute_tail(0, False)
    fire_scatters(0)

    do_chunk(neg_flat, jnp.int32(0), 1, 2, True, nxt=jnp.int32(1))
    do_chunk(neg_flat, jnp.int32(1), 2, 0, True, nxt=jnp.int32(2))

    @pl.loop(0, 45, step=_NBUF)
    def _main2(i):
        for b in range(_NBUF):
            ll = 2 + i + b
            do_chunk(neg_flat, ll, b, (b + 1) % _NBUF, True, nxt=ll + 1)

    do_chunk(neg_flat, jnp.int32(47), 0, 1, True, nxt=None)
    fire_in(neg_flat, jnp.int32(_NFULL), 1, tail=True)

    wait_in(neg_flat, jnp.int32(_NFULL), 1, tail=True)
    drain_scatters(2)
    compute_tail(1, True)
    fire_scatters(1)

    drain_scatters(0)
    drain_scatters(1)

    plsc.subcore_barrier()

    num_out = [num0, num1][:_NCORES]
    den_out = [den0, den1][:_NCORES]
    for core in range(_NCORES):
        @pl.when(cid == core)
        def _():
            for acc, out in ((acc_num, num_out[core]),
                             (acc_den, den_out[core])):
                for off, ln in _PIECES:
                    pltpu.sync_copy(acc.at[pl.ds(sid * _SL + off, ln)],
                                    stage_v.at[pl.ds(0, ln)])
                    pltpu.sync_copy(stage_v.at[pl.ds(0, ln)],
                                    out.at[pl.ds(sid * _SL + off, ln)])


def _make_sc_kernel():
    mesh = plsc.VectorSubcoreMesh(
        core_axis_name="c", subcore_axis_name="s",
        num_cores=_NCORES, num_subcores=_NSUB)
    out = jax.ShapeDtypeStruct((_NC_PAD,), jnp.float32)
    return pl.kernel(
        _sc_kernel_body,
        out_type=(out, out, out, out),
        mesh=mesh,
        compiler_params=pltpu.CompilerParams(needs_layout_passes=False),
        scratch_types=[
            pltpu.VMEM((_N_VARS,), jnp.float32),
            *([pltpu.VMEM((_K, 128), jnp.int32)] * _NBUF),
            *([pltpu.VMEM((_C,), jnp.int32)] * _NBUF),
            *([pltpu.VMEM((_C,), jnp.float32)] * _NBUF),
            *([pltpu.VMEM((_C,), jnp.float32)] * _NBUF),
            pltpu.VMEM((_C,), jnp.float32),
            pltpu.VMEM_SHARED((_NC_PAD,), jnp.float32),
            pltpu.VMEM_SHARED((_NC_PAD,), jnp.float32),
            pltpu.SemaphoreType.DMA,
            pltpu.SemaphoreType.DMA,
            pltpu.SemaphoreType.DMA,
            pltpu.SemaphoreType.DMA,
            pltpu.SemaphoreType.DMA,
            pltpu.SemaphoreType.DMA,
        ],
    )


def _fin_body(*refs):
    out_ref = refs[-1]
    cc_ref = refs[-2]
    half = (len(refs) - 2) // 2
    num = sum(r[...] for r in refs[:half])
    den = sum(r[...] for r in refs[half:2 * half])
    sm = 1.0 / (1.0 + jnp.exp(_A * (0.5 - num / den)))
    row = lax.broadcasted_iota(jnp.int32, (_R, 128), 0)
    col = lax.broadcasted_iota(jnp.int32, (_R, 128), 1)
    mask = (row * 128 + col) < _N_CLAUSES
    diff = jnp.where(mask, sm - cc_ref[...], 0.0)
    out_ref[0, 0] = jnp.sum(diff * diff) / _N_CLAUSES


def _finalize(ns, ds, cc_pad):
    shape2d = (_R, 128)
    args = [a.reshape(shape2d) for a in (*ns, *ds, cc_pad)]
    loss = pl.pallas_call(
        _fin_body,
        out_shape=jax.ShapeDtypeStruct((1, 1), jnp.float32),
        in_specs=[pl.BlockSpec(memory_space=pltpu.VMEM)] * len(args),
        out_specs=pl.BlockSpec(memory_space=pltpu.SMEM),
    )(*args)
    return loss[0, 0]


def kernel(xv, adj_pos, adj_neg, clause_count, is_train):
    del is_train
    xvf = xv.reshape(-1)
    pos_flat = adj_pos.reshape(-1)
    neg_flat = adj_neg.reshape(-1)

    outs = _make_sc_kernel()(xvf, pos_flat, neg_flat)
    ns = [outs[2 * i] for i in range(_NCORES)]
    ds = [outs[2 * i + 1] for i in range(_NCORES)]

    cc_pad = jnp.pad(clause_count, (0, _NC_PAD - _N_CLAUSES))
    return _finalize(ns, ds, cc_pad)
